# Initial kernel scaffold; baseline (speedup 1.0000x reference)
import jax
import jax.numpy as jnp
from jax.experimental import pallas as pl


def kernel(hidden_states, router_weight, merged_gate_up_proj, merged_down_proj):
    raise NotImplementedError("write your pallas kernel here")


# speedup vs baseline: 1.3790x; 1.3790x over previous
import functools

import jax
import jax.numpy as jnp
from jax.experimental import pallas as pl
from jax.experimental.pallas import tpu as pltpu

E = 8
TOP_K = 2
D = 1024
FF = 1408
T = 4096

TB = 2048
NT = T // TB
FC = 128
NF = FF // FC


def _moe_body(x_ref, rw_ref, wg_ref, wu_ref, wd_ref, out_ref, we_s):
    e = pl.program_id(1)
    f = pl.program_id(2)

    @pl.when(jnp.logical_and(e == 0, f == 0))
    def _router():
        x = x_ref[...]
        logits = jax.lax.dot_general(
            x, rw_ref[...], (((1,), (1,)), ((), ())),
            preferred_element_type=jnp.float32)
        m = jnp.max(logits, axis=-1, keepdims=True)
        ex = jnp.exp(logits - m)
        probs = ex / jnp.sum(ex, axis=-1, keepdims=True)
        a1 = jnp.argmax(probs, axis=-1)
        p1 = jnp.max(probs, axis=-1)
        cols = jax.lax.broadcasted_iota(jnp.int32, probs.shape, 1)
        masked = jnp.where(cols == a1[:, None], -jnp.inf, probs)
        a2 = jnp.argmax(masked, axis=-1)
        p2 = jnp.max(masked, axis=-1)
        we = (jnp.where(cols == a1[:, None], p1[:, None], 0.0)
              + jnp.where(cols == a2[:, None], p2[:, None], 0.0))
        we_s[...] = we

    x = x_ref[...]
    gate = jnp.dot(x, wg_ref[0], preferred_element_type=jnp.float32)
    up = jnp.dot(x, wu_ref[0], preferred_element_type=jnp.float32)
    h = gate * jax.lax.logistic(gate) * up
    we = we_s[...]
    ecols = jax.lax.broadcasted_iota(jnp.int32, we.shape, 1)
    w_col = jnp.sum(jnp.where(ecols == e, we, 0.0), axis=-1, keepdims=True)
    part = jnp.dot(h * w_col, wd_ref[0], preferred_element_type=jnp.float32)

    @pl.when(jnp.logical_and(e == 0, f == 0))
    def _init():
        out_ref[...] = part

    @pl.when(jnp.logical_not(jnp.logical_and(e == 0, f == 0)))
    def _acc():
        out_ref[...] += part


def kernel(hidden_states, router_weight, merged_gate_up_proj, merged_down_proj):
    grid = (NT, E, NF)
    out = pl.pallas_call(
        _moe_body,
        grid=grid,
        in_specs=[
            pl.BlockSpec((TB, D), lambda t, e, f: (t, 0)),
            pl.BlockSpec((E, D), lambda t, e, f: (0, 0)),
            pl.BlockSpec((1, D, FC), lambda t, e, f: (e, 0, f)),
            pl.BlockSpec((1, D, FC), lambda t, e, f: (e, 0, f + NF)),
            pl.BlockSpec((1, FC, D), lambda t, e, f: (e, f, 0)),
        ],
        out_specs=pl.BlockSpec((TB, D), lambda t, e, f: (t, 0)),
        out_shape=jax.ShapeDtypeStruct((T, D), jnp.float32),
        scratch_shapes=[pltpu.VMEM((TB, E), jnp.float32)],
    )(hidden_states, router_weight, merged_gate_up_proj,
      merged_gate_up_proj, merged_down_proj)
    return out


# speedup vs baseline: 1.6435x; 1.1918x over previous
import functools

import numpy as np
import jax
import jax.numpy as jnp
from jax import lax
from jax.experimental import pallas as pl
from jax.experimental.pallas import tpu as pltpu
from jax.experimental.pallas import tpu_sc as plsc

E = 8
K = 2
D = 1024
FF = 1408
T = 4096
R = T * K

RB = 128
NRB = T // RB

BLK = 256
NBLKS = R // BLK + E
P = NBLKS * BLK
FC = 128
NF = FF // FC

NW = 32
CS = 32
NCH = R // NW // CS
TW = T // NW
TG = 16


def _router_body(x_ref, rw_ref, e01_ref, rank_ref, wrep_ref, cum_ref, carry):
    c = pl.program_id(0)

    @pl.when(c == 0)
    def _init():
        carry[...] = jnp.zeros_like(carry)

    x = x_ref[...]
    logits = lax.dot_general(x, rw_ref[...], (((1,), (1,)), ((), ())),
                             preferred_element_type=jnp.float32)
    m = jnp.max(logits, axis=-1, keepdims=True)
    ex = jnp.exp(logits - m)
    probs = ex / jnp.sum(ex, axis=-1, keepdims=True)

    iota8 = lax.broadcasted_iota(jnp.int32, (RB, E), 1)
    a1 = jnp.argmax(probs, axis=-1)
    p1 = jnp.max(probs, axis=-1)
    masked = jnp.where(iota8 == a1[:, None], -1.0, probs)
    a2 = jnp.argmax(masked, axis=-1)
    p2 = jnp.max(masked, axis=-1)

    lane32 = lax.broadcasted_iota(jnp.int32, (RB, 32), 1)
    wrep_ref[...] = jnp.where(lane32 < 16, p1[:, None], p2[:, None])
    e01_ref[...] = jnp.concatenate(
        [a1[:, None], a2[:, None]], axis=1).astype(jnp.int32)

    h0 = (iota8 == a1[:, None]).astype(jnp.float32)
    h1 = (iota8 == a2[:, None]).astype(jnp.float32)
    s = h0 + h1
    ri = lax.broadcasted_iota(jnp.int32, (RB, RB), 0)
    ci = lax.broadcasted_iota(jnp.int32, (RB, RB), 1)
    lstrict = (ri > ci).astype(jnp.float32)
    pex = lax.dot_general(lstrict, s, (((1,), (0,)), ((), ())),
                          preferred_element_type=jnp.float32)
    base = carry[...]
    msum = pex + base
    r0 = jnp.sum(jnp.where(iota8 == a1[:, None], msum, 0.0), axis=-1)
    r1 = jnp.sum(jnp.where(iota8 == a2[:, None], msum + h0, 0.0), axis=-1)
    rank_ref[...] = jnp.concatenate(
        [r0[:, None], r1[:, None]], axis=1).astype(jnp.int32)
    newc = base + jnp.sum(s, axis=0, keepdims=True)
    carry[...] = newc
    cum_ref[...] = newc[None].astype(jnp.int32)


def _finalize_body(e01_ref, rank_ref, cum_ref, pos_ref, beo_ref):
    counts = cum_ref[NRB - 1].astype(jnp.float32)
    nb = jnp.ceil(counts / BLK)
    er = lax.broadcasted_iota(jnp.int32, (E, E), 0)
    ec = lax.broadcasted_iota(jnp.int32, (E, E), 1)
    uinc = (er <= ec).astype(jnp.float32)
    cuminc = lax.dot_general(nb, uinc, (((1,), (0,)), ((), ())),
                             preferred_element_type=jnp.float32)
    start = (cuminc - nb) * float(BLK)

    e01 = e01_ref[...]
    rank = rank_ref[...]
    iota8r = lax.broadcasted_iota(jnp.int32, (T, E), 1)
    s0 = jnp.sum(jnp.where(iota8r == e01[:, 0:1], start, 0.0), axis=-1)
    s1 = jnp.sum(jnp.where(iota8r == e01[:, 1:2], start, 0.0), axis=-1)
    pos0 = rank[:, 0] + s0.astype(jnp.int32)
    pos1 = rank[:, 1] + s1.astype(jnp.int32)
    pos_ref[...] = jnp.concatenate([pos0[:, None], pos1[:, None]], axis=1)

    nb1 = NBLKS + 1
    bcol = lax.broadcasted_iota(jnp.int32, (nb1, E), 0)
    cmp = (cuminc.astype(jnp.int32) <= bcol).astype(jnp.int32)
    be = jnp.minimum(jnp.sum(cmp, axis=-1), E - 1)
    nused = jnp.sum(nb, dtype=jnp.float32).astype(jnp.int32)
    lanei = lax.broadcasted_iota(jnp.int32, (1, nb1), 1)
    beo_ref[...] = jnp.where(lanei == NBLKS, nused, be[None, :])


def _gemm_body(be_ref, x_ref, wg_ref, wu_ref, wd_ref, o_ref):
    b = pl.program_id(0)
    f = pl.program_id(1)
    nused = be_ref[NBLKS]

    @pl.when(b < nused)
    def _compute():
        x = x_ref[...].astype(jnp.bfloat16)
        gate = jnp.dot(x, wg_ref[0], preferred_element_type=jnp.float32)
        up = jnp.dot(x, wu_ref[0], preferred_element_type=jnp.float32)
        h = gate * lax.logistic(gate) * up
        part = jnp.dot(h.astype(jnp.bfloat16), wd_ref[0],
                       preferred_element_type=jnp.float32)

        @pl.when(f == 0)
        def _first():
            o_ref[...] = part

        @pl.when(f > 0)
        def _acc():
            o_ref[...] += part


def _make_dispatch():
    mesh = plsc.VectorSubcoreMesh(core_axis_name="c", subcore_axis_name="s")

    @functools.partial(
        pl.kernel, mesh=mesh,
        out_type=jax.ShapeDtypeStruct((P, D), jnp.float32),
        scratch_types=[
            pltpu.VMEM((NCH, CS), jnp.int32),
            pltpu.VMEM((NCH, CS), jnp.int32),
            pltpu.VMEM((CS, D), jnp.float32),
            pltpu.VMEM((CS, D), jnp.float32),
            pltpu.SemaphoreType.DMA,
            pltpu.SemaphoreType.DMA,
            pltpu.SemaphoreType.DMA,
        ],
    )
    def dispatch(hid_hbm, tok_hbm, pos_hbm, perm_hbm,
                 tok_v, pos_v, rows_a, rows_b, gsem_a, gsem_b, ssem):
        wid = lax.axis_index("s") * 2 + lax.axis_index("c")
        pltpu.sync_copy(tok_hbm.at[wid], tok_v)
        pltpu.sync_copy(pos_hbm.at[wid], pos_v)
        bufs = (rows_a, rows_b)
        sems = (gsem_a, gsem_b)
        pending = [None, None]
        pending[0] = pltpu.async_copy(hid_hbm.at[tok_v.at[0]], bufs[0], sems[0])
        for j in range(NCH):
            cur = j % 2
            pending[cur].wait()
            if j + 1 < NCH:
                nxt = (j + 1) % 2
                pending[nxt] = pltpu.async_copy(
                    hid_hbm.at[tok_v.at[j + 1]], bufs[nxt], sems[nxt])
            pltpu.async_copy(bufs[cur], perm_hbm.at[pos_v.at[j]], ssem).wait()

    return dispatch


def _make_combine():
    mesh = plsc.VectorSubcoreMesh(core_axis_name="c", subcore_axis_name="s")

    @functools.partial(
        pl.kernel, mesh=mesh,
        out_type=jax.ShapeDtypeStruct((T, D), jnp.float32),
        scratch_types=[
            pltpu.VMEM((NCH, CS), jnp.int32),
            pltpu.VMEM((TW, 32), jnp.float32),
            pltpu.VMEM((CS, D), jnp.float32),
            pltpu.VMEM((CS, D), jnp.float32),
            pltpu.VMEM((TG, D), jnp.float32),
            pltpu.SemaphoreType.DMA,
            pltpu.SemaphoreType.DMA,
            pltpu.SemaphoreType.DMA,
        ],
    )
    def combine(operm_hbm, pos_hbm, wrep_hbm, out_hbm,
                pos_v, w_v, rows_a, rows_b, o_v, gsem_a, gsem_b, ssem):
        wid = lax.axis_index("s") * 2 + lax.axis_index("c")
        pltpu.sync_copy(pos_hbm.at[wid], pos_v)
        pltpu.sync_copy(wrep_hbm.at[pl.ds(wid * TW, TW)], w_v)
        bufs = (rows_a, rows_b)
        sems = (gsem_a, gsem_b)
        pending = [None, None]
        pending[0] = pltpu.async_copy(
            operm_hbm.at[pos_v.at[0]], bufs[0], sems[0])
        for j in range(NCH):
            cur = j % 2
            pending[cur].wait()
            if j + 1 < NCH:
                nxt = (j + 1) % 2
                pending[nxt] = pltpu.async_copy(
                    operm_hbm.at[pos_v.at[j + 1]], bufs[nxt], sems[nxt])
            rows = bufs[cur]
            for tt in range(TG):
                w0 = w_v[j * TG + tt, 0:16]
                w1 = w_v[j * TG + tt, 16:32]

                def cbody(ci, _, tt=tt, w0=w0, w1=w1, rows=rows):
                    sl = pl.ds(ci * 16, 16)
                    o_v[tt, sl] = w0 * rows[2 * tt, sl] + w1 * rows[2 * tt + 1, sl]
                    return 0

                lax.fori_loop(0, D // 16, cbody, 0)
            pltpu.sync_copy(o_v, out_hbm.at[pl.ds(wid * TW + j * TG, TG)])

    return combine


_TOK3 = np.repeat(np.arange(T, dtype=np.int32), K).reshape(NW, NCH, CS)


def kernel(hidden_states, router_weight, merged_gate_up_proj, merged_down_proj):
    e01, rank, wrep, cum = pl.pallas_call(
        _router_body,
        grid=(NRB,),
        in_specs=[
            pl.BlockSpec((RB, D), lambda c: (c, 0)),
            pl.BlockSpec((E, D), lambda c: (0, 0)),
        ],
        out_specs=[
            pl.BlockSpec((RB, K), lambda c: (c, 0)),
            pl.BlockSpec((RB, K), lambda c: (c, 0)),
            pl.BlockSpec((RB, 32), lambda c: (c, 0)),
            pl.BlockSpec((1, 1, E), lambda c: (c, 0, 0)),
        ],
        out_shape=[
            jax.ShapeDtypeStruct((T, K), jnp.int32),
            jax.ShapeDtypeStruct((T, K), jnp.int32),
            jax.ShapeDtypeStruct((T, 32), jnp.float32),
            jax.ShapeDtypeStruct((NRB, 1, E), jnp.int32),
        ],
        scratch_shapes=[pltpu.VMEM((1, E), jnp.float32)],
    )(hidden_states, router_weight)

    pos, beo = pl.pallas_call(
        _finalize_body,
        grid=(1,),
        in_specs=[
            pl.BlockSpec((T, K), lambda i: (0, 0)),
            pl.BlockSpec((T, K), lambda i: (0, 0)),
            pl.BlockSpec((NRB, 1, E), lambda i: (0, 0, 0)),
        ],
        out_specs=[
            pl.BlockSpec((T, K), lambda i: (0, 0)),
            pl.BlockSpec((1, NBLKS + 1), lambda i: (0, 0)),
        ],
        out_shape=[
            jax.ShapeDtypeStruct((T, K), jnp.int32),
            jax.ShapeDtypeStruct((1, NBLKS + 1), jnp.int32),
        ],
    )(e01, rank, cum)

    pos3 = pos.reshape(NW, NCH, CS)
    tok3 = jnp.asarray(_TOK3)
    beo_flat = beo.reshape(NBLKS + 1)

    perm = _make_dispatch()(hidden_states, tok3, pos3)

    gu_bf = merged_gate_up_proj.astype(jnp.bfloat16)
    dn_bf = merged_down_proj.astype(jnp.bfloat16)

    grid_spec = pltpu.PrefetchScalarGridSpec(
        num_scalar_prefetch=1,
        grid=(NBLKS, NF),
        in_specs=[
            pl.BlockSpec((BLK, D), lambda b, f, be: (b, 0)),
            pl.BlockSpec((1, D, FC), lambda b, f, be: (be[b], 0, f)),
            pl.BlockSpec((1, D, FC), lambda b, f, be: (be[b], 0, f + NF)),
            pl.BlockSpec((1, FC, D), lambda b, f, be: (be[b], f, 0)),
        ],
        out_specs=pl.BlockSpec((BLK, D), lambda b, f, be: (b, 0)),
    )
    out_perm = pl.pallas_call(
        _gemm_body,
        grid_spec=grid_spec,
        out_shape=jax.ShapeDtypeStruct((P, D), jnp.float32),
    )(beo_flat, perm, gu_bf, gu_bf, dn_bf)

    combined = _make_combine()(out_perm, pos3, wrep)
    return combined


# speedup vs baseline: 4.0354x; 2.4553x over previous
import functools

import numpy as np
import jax
import jax.numpy as jnp
from jax import lax
from jax.experimental import pallas as pl
from jax.experimental.pallas import tpu as pltpu
from jax.experimental.pallas import tpu_sc as plsc

E = 8
K = 2
D = 1024
FF = 1408
T = 4096
R = T * K

RB = 512
NRB = T // RB

BLK = 256
NBLKS = R // BLK + E
P = NBLKS * BLK

NW = 32
CS = 32
NCH = R // NW // CS
TW = T // NW
TG = 16


def _router_body(x_ref, rw_ref, e01_ref, rank_ref, wrep_ref, cum_ref, carry):
    c = pl.program_id(0)

    @pl.when(c == 0)
    def _init():
        carry[...] = jnp.zeros_like(carry)

    x = x_ref[...]
    logits = lax.dot_general(x, rw_ref[...], (((1,), (1,)), ((), ())),
                             preferred_element_type=jnp.float32)
    m = jnp.max(logits, axis=-1, keepdims=True)
    ex = jnp.exp(logits - m)
    probs = ex / jnp.sum(ex, axis=-1, keepdims=True)

    iota8 = lax.broadcasted_iota(jnp.int32, (RB, E), 1)
    a1 = jnp.argmax(probs, axis=-1)
    p1 = jnp.max(probs, axis=-1)
    masked = jnp.where(iota8 == a1[:, None], -1.0, probs)
    a2 = jnp.argmax(masked, axis=-1)
    p2 = jnp.max(masked, axis=-1)

    lane32 = lax.broadcasted_iota(jnp.int32, (RB, 32), 1)
    wrep_ref[...] = jnp.where(lane32 < 16, p1[:, None], p2[:, None])
    e01_ref[...] = jnp.concatenate(
        [a1[:, None], a2[:, None]], axis=1).astype(jnp.int32)

    h0 = (iota8 == a1[:, None]).astype(jnp.float32)
    h1 = (iota8 == a2[:, None]).astype(jnp.float32)
    s = h0 + h1
    ri = lax.broadcasted_iota(jnp.int32, (RB, RB), 0)
    ci = lax.broadcasted_iota(jnp.int32, (RB, RB), 1)
    lstrict = (ri > ci).astype(jnp.float32)
    pex = lax.dot_general(lstrict, s, (((1,), (0,)), ((), ())),
                          preferred_element_type=jnp.float32)
    base = carry[...]
    msum = pex + base
    r0 = jnp.sum(jnp.where(iota8 == a1[:, None], msum, 0.0), axis=-1)
    r1 = jnp.sum(jnp.where(iota8 == a2[:, None], msum + h0, 0.0), axis=-1)
    rank_ref[...] = jnp.concatenate(
        [r0[:, None], r1[:, None]], axis=1).astype(jnp.int32)
    newc = base + jnp.sum(s, axis=0, keepdims=True)
    carry[...] = newc
    cum_ref[...] = newc[None].astype(jnp.int32)


def _finalize_body(e01_ref, rank_ref, cum_ref, pos_ref, beo_ref):
    counts = cum_ref[NRB - 1].astype(jnp.float32)
    nb = jnp.ceil(counts / BLK)
    er = lax.broadcasted_iota(jnp.int32, (E, E), 0)
    ec = lax.broadcasted_iota(jnp.int32, (E, E), 1)
    uinc = (er <= ec).astype(jnp.float32)
    cuminc = lax.dot_general(nb, uinc, (((1,), (0,)), ((), ())),
                             preferred_element_type=jnp.float32)
    start = (cuminc - nb) * float(BLK)

    e01 = e01_ref[...]
    rank = rank_ref[...]
    iota8r = lax.broadcasted_iota(jnp.int32, (T, E), 1)
    s0 = jnp.sum(jnp.where(iota8r == e01[:, 0:1], start, 0.0), axis=-1)
    s1 = jnp.sum(jnp.where(iota8r == e01[:, 1:2], start, 0.0), axis=-1)
    pos0 = rank[:, 0] + s0.astype(jnp.int32)
    pos1 = rank[:, 1] + s1.astype(jnp.int32)
    pos_ref[...] = jnp.concatenate([pos0[:, None], pos1[:, None]], axis=1)

    nb1 = NBLKS + 1
    bcol = lax.broadcasted_iota(jnp.int32, (nb1, E), 0)
    cmp = (cuminc.astype(jnp.int32) <= bcol).astype(jnp.int32)
    be = jnp.minimum(jnp.sum(cmp, axis=-1), E - 1)
    nused = jnp.sum(nb, dtype=jnp.float32).astype(jnp.int32)
    lanei = lax.broadcasted_iota(jnp.int32, (1, nb1), 1)
    beo_ref[...] = jnp.where(lanei == NBLKS, nused, be[None, :])


def _gemm_body(be_ref, x_ref, wg_ref, wu_ref, wd_ref, o_ref):
    b = pl.program_id(0)
    nused = be_ref[NBLKS]

    @pl.when(b < nused)
    def _compute():
        x = x_ref[...]
        gate = jnp.dot(x, wg_ref[0], preferred_element_type=jnp.float32)
        up = jnp.dot(x, wu_ref[0], preferred_element_type=jnp.float32)
        h = gate * lax.logistic(gate) * up
        o_ref[...] = jnp.dot(h, wd_ref[0], preferred_element_type=jnp.float32)


def _make_dispatch():
    mesh = plsc.VectorSubcoreMesh(core_axis_name="c", subcore_axis_name="s")

    @functools.partial(
        pl.kernel, mesh=mesh,
        out_type=jax.ShapeDtypeStruct((P, D), jnp.float32),
        scratch_types=[
            pltpu.VMEM((NCH, CS), jnp.int32),
            pltpu.VMEM((NCH, CS), jnp.int32),
            pltpu.VMEM((CS, D), jnp.float32),
            pltpu.VMEM((CS, D), jnp.float32),
            pltpu.SemaphoreType.DMA,
            pltpu.SemaphoreType.DMA,
            pltpu.SemaphoreType.DMA,
        ],
    )
    def dispatch(hid_hbm, tok_hbm, pos_hbm, perm_hbm,
                 tok_v, pos_v, rows_a, rows_b, gsem_a, gsem_b, ssem):
        wid = lax.axis_index("s") * 2 + lax.axis_index("c")
        pltpu.sync_copy(tok_hbm.at[wid], tok_v)
        pltpu.sync_copy(pos_hbm.at[wid], pos_v)
        bufs = (rows_a, rows_b)
        sems = (gsem_a, gsem_b)
        pending = [None, None]
        pending[0] = pltpu.async_copy(hid_hbm.at[tok_v.at[0]], bufs[0], sems[0])
        for j in range(NCH):
            cur = j % 2
            pending[cur].wait()
            if j + 1 < NCH:
                nxt = (j + 1) % 2
                pending[nxt] = pltpu.async_copy(
                    hid_hbm.at[tok_v.at[j + 1]], bufs[nxt], sems[nxt])
            pltpu.async_copy(bufs[cur], perm_hbm.at[pos_v.at[j]], ssem).wait()

    return dispatch


def _make_combine():
    mesh = plsc.VectorSubcoreMesh(core_axis_name="c", subcore_axis_name="s")

    @functools.partial(
        pl.kernel, mesh=mesh,
        out_type=jax.ShapeDtypeStruct((T, D), jnp.float32),
        scratch_types=[
            pltpu.VMEM((NCH, CS), jnp.int32),
            pltpu.VMEM((TW, 32), jnp.float32),
            pltpu.VMEM((CS, D), jnp.float32),
            pltpu.VMEM((CS, D), jnp.float32),
            pltpu.VMEM((TG, D), jnp.float32),
            pltpu.SemaphoreType.DMA,
            pltpu.SemaphoreType.DMA,
            pltpu.SemaphoreType.DMA,
        ],
    )
    def combine(operm_hbm, pos_hbm, wrep_hbm, out_hbm,
                pos_v, w_v, rows_a, rows_b, o_v, gsem_a, gsem_b, ssem):
        wid = lax.axis_index("s") * 2 + lax.axis_index("c")
        pltpu.sync_copy(pos_hbm.at[wid], pos_v)
        pltpu.sync_copy(wrep_hbm.at[pl.ds(wid * TW, TW)], w_v)
        bufs = (rows_a, rows_b)
        sems = (gsem_a, gsem_b)
        pending = [None, None]
        pending[0] = pltpu.async_copy(
            operm_hbm.at[pos_v.at[0]], bufs[0], sems[0])
        for j in range(NCH):
            cur = j % 2
            pending[cur].wait()
            if j + 1 < NCH:
                nxt = (j + 1) % 2
                pending[nxt] = pltpu.async_copy(
                    operm_hbm.at[pos_v.at[j + 1]], bufs[nxt], sems[nxt])
            rows = bufs[cur]
            for tt in range(TG):
                w0 = w_v[j * TG + tt, 0:16]
                w1 = w_v[j * TG + tt, 16:32]

                def cbody(ci, _, tt=tt, w0=w0, w1=w1, rows=rows):
                    sl = pl.ds(ci * 16, 16)
                    o_v[tt, sl] = w0 * rows[2 * tt, sl] + w1 * rows[2 * tt + 1, sl]
                    return 0

                lax.fori_loop(0, D // 16, cbody, 0)
            pltpu.sync_copy(o_v, out_hbm.at[pl.ds(wid * TW + j * TG, TG)])

    return combine


_TOK3 = np.repeat(np.arange(T, dtype=np.int32), K).reshape(NW, NCH, CS)


def kernel(hidden_states, router_weight, merged_gate_up_proj, merged_down_proj):
    e01, rank, wrep, cum = pl.pallas_call(
        _router_body,
        grid=(NRB,),
        in_specs=[
            pl.BlockSpec((RB, D), lambda c: (c, 0)),
            pl.BlockSpec((E, D), lambda c: (0, 0)),
        ],
        out_specs=[
            pl.BlockSpec((RB, K), lambda c: (c, 0)),
            pl.BlockSpec((RB, K), lambda c: (c, 0)),
            pl.BlockSpec((RB, 32), lambda c: (c, 0)),
            pl.BlockSpec((1, 1, E), lambda c: (c, 0, 0)),
        ],
        out_shape=[
            jax.ShapeDtypeStruct((T, K), jnp.int32),
            jax.ShapeDtypeStruct((T, K), jnp.int32),
            jax.ShapeDtypeStruct((T, 32), jnp.float32),
            jax.ShapeDtypeStruct((NRB, 1, E), jnp.int32),
        ],
        scratch_shapes=[pltpu.VMEM((1, E), jnp.float32)],
    )(hidden_states, router_weight)

    pos, beo = pl.pallas_call(
        _finalize_body,
        grid=(1,),
        in_specs=[
            pl.BlockSpec((T, K), lambda i: (0, 0)),
            pl.BlockSpec((T, K), lambda i: (0, 0)),
            pl.BlockSpec((NRB, 1, E), lambda i: (0, 0, 0)),
        ],
        out_specs=[
            pl.BlockSpec((T, K), lambda i: (0, 0)),
            pl.BlockSpec((1, NBLKS + 1), lambda i: (0, 0)),
        ],
        out_shape=[
            jax.ShapeDtypeStruct((T, K), jnp.int32),
            jax.ShapeDtypeStruct((1, NBLKS + 1), jnp.int32),
        ],
    )(e01, rank, cum)

    pos3 = pos.reshape(NW, NCH, CS)
    tok3 = jnp.asarray(_TOK3)
    beo_flat = beo.reshape(NBLKS + 1)

    perm = _make_dispatch()(hidden_states, tok3, pos3)

    grid_spec = pltpu.PrefetchScalarGridSpec(
        num_scalar_prefetch=1,
        grid=(NBLKS,),
        in_specs=[
            pl.BlockSpec((BLK, D), lambda b, be: (b, 0)),
            pl.BlockSpec((1, D, FF), lambda b, be: (be[b], 0, 0)),
            pl.BlockSpec((1, D, FF), lambda b, be: (be[b], 0, 1)),
            pl.BlockSpec((1, FF, D), lambda b, be: (be[b], 0, 0)),
        ],
        out_specs=pl.BlockSpec((BLK, D), lambda b, be: (b, 0)),
    )
    out_perm = pl.pallas_call(
        _gemm_body,
        grid_spec=grid_spec,
        out_shape=jax.ShapeDtypeStruct((P, D), jnp.float32),
    )(beo_flat, perm, merged_gate_up_proj, merged_gate_up_proj,
      merged_down_proj)

    combined = _make_combine()(out_perm, pos3, wrep)
    return combined
